# baseline (speedup 1.0000x reference)
import functools
import math

import jax
import jax.numpy as jnp
from jax import lax
from jax.experimental import pallas as pl
from jax.experimental.pallas import tpu as pltpu
from jax.experimental.pallas import tpu_sc as plsc

NC = 2
NS = 16
NW = NC * NS
L = 16

D = 64
B = 4096 * 200
W = 128
B_PER_W = B // NW
NCH = B_PER_W // W
SCALE = math.sqrt(D)

_mesh = plsc.VectorSubcoreMesh(core_axis_name="c", subcore_axis_name="s")


@functools.partial(
    pl.kernel,
    out_type=jax.ShapeDtypeStruct((B, D), jnp.float32),
    mesh=_mesh,
    scratch_types=[
        pltpu.VMEM((NCH, W), jnp.int32),
        pltpu.VMEM((W, D), jnp.float32),
    ],
    compiler_params=pltpu.CompilerParams(use_tc_tiling_on_sc=False),
)
def _gather_scale(lut_hbm, xi_hbm, out_hbm, idx_v, rows_v):
    wid = lax.axis_index("s") * NC + lax.axis_index("c")
    ch0 = wid * NCH
    pltpu.sync_copy(xi_hbm.at[pl.ds(ch0, NCH)], idx_v)

    @pl.loop(0, NCH)
    def _(c):
        pltpu.sync_copy(lut_hbm.at[idx_v.at[c]], rows_v)

        @pl.loop(0, W)
        def _(r):
            for col in range(0, D, L):
                rows_v.at[r, pl.ds(col, L)][...] = (
                    rows_v.at[r, pl.ds(col, L)][...] * SCALE
                )

        pltpu.sync_copy(rows_v, out_hbm.at[pl.ds((ch0 + c) * W, W)])


def kernel(x, lut):
    xi = x.reshape(B // W, W).astype(jnp.int32)
    out = _gather_scale(lut, xi)
    return out.reshape(x.shape[0], x.shape[1], D)


# speedup vs baseline: 1.2098x; 1.2098x over previous
import functools
import math

import jax
import jax.numpy as jnp
from jax import lax
from jax.experimental import pallas as pl
from jax.experimental.pallas import tpu as pltpu
from jax.experimental.pallas import tpu_sc as plsc

NC = 2
NS = 16
NW = NC * NS
L = 16

D = 64
B = 4096 * 200
W = 128
NB = 4
B_PER_W = B // NW
NCH = B_PER_W // W
SCALE = math.sqrt(D)

_mesh = plsc.VectorSubcoreMesh(core_axis_name="c", subcore_axis_name="s")


@functools.partial(
    pl.kernel,
    out_type=jax.ShapeDtypeStruct((B, D), jnp.float32),
    mesh=_mesh,
    scratch_types=[
        pltpu.VMEM((NCH, W), jnp.int32),
        pltpu.VMEM((NB, W, D), jnp.float32),
        pltpu.VMEM((NB, W, D), jnp.float32),
        pltpu.SemaphoreType.DMA((NB,)),
        pltpu.SemaphoreType.DMA((NB,)),
    ],
    compiler_params=pltpu.CompilerParams(use_tc_tiling_on_sc=False),
)
def _gather_scale(lut_hbm, xi_hbm, out_hbm, idx_v, rows_g, rows_o, gsem, osem):
    wid = lax.axis_index("s") * NC + lax.axis_index("c")
    ch0 = wid * NCH
    pltpu.sync_copy(xi_hbm.at[pl.ds(ch0, NCH)], idx_v)

    for b in range(NB):
        pltpu.make_async_copy(
            lut_hbm.at[idx_v.at[b]], rows_g.at[b], gsem.at[b]
        ).start()

    @pl.loop(0, NCH, step=NB)
    def _(c0):
        for b in range(NB):
            c = c0 + b
            pltpu.make_async_copy(
                lut_hbm.at[idx_v.at[c]], rows_g.at[b], gsem.at[b]
            ).wait()

            @pl.when(c0 > 0)
            def _():
                pltpu.make_async_copy(
                    rows_o.at[b],
                    out_hbm.at[pl.ds((ch0 + c - NB) * W, W)],
                    osem.at[b],
                ).wait()

            @pl.loop(0, W)
            def _(r):
                for col in range(0, D, L):
                    rows_o.at[b, r, pl.ds(col, L)][...] = (
                        rows_g.at[b, r, pl.ds(col, L)][...] * SCALE
                    )

            pltpu.make_async_copy(
                rows_o.at[b], out_hbm.at[pl.ds((ch0 + c) * W, W)], osem.at[b]
            ).start()

            @pl.when(c0 + NB < NCH)
            def _():
                pltpu.make_async_copy(
                    lut_hbm.at[idx_v.at[c + NB]], rows_g.at[b], gsem.at[b]
                ).start()

    for b in range(NB):
        pltpu.make_async_copy(
            rows_o.at[b],
            out_hbm.at[pl.ds((ch0 + NCH - NB + b) * W, W)],
            osem.at[b],
        ).wait()


def kernel(x, lut):
    xi = x.reshape(B // W, W).astype(jnp.int32)
    out = _gather_scale(lut, xi)
    return out.reshape(x.shape[0], x.shape[1], D)
